# Initial kernel scaffold; baseline (speedup 1.0000x reference)
import jax
import jax.numpy as jnp
from jax import lax
from jax.experimental import pallas as pl
from jax.experimental.pallas import tpu as pltpu
from jax.experimental.pallas import tpu_sc as plsc

VOCAB = 1000000
D = 32
ROWS = 16384
COLS = 200
B = ROWS * COLS
NC = 2
NS = 16
NW = NC * NS
B_PER_W = B // NW
C = 1024
NCHUNK = B_PER_W // C
SCALE = float(D) ** 0.5


def _body(x_hbm, table_hbm, out_hbm, idx_v, rows_v, gsem):
    wid = lax.axis_index("s") * NC + lax.axis_index("c")
    base = wid * B_PER_W

    @pl.loop(0, NCHUNK)
    def _chunk(g):
        off = base + g * C
        pltpu.sync_copy(x_hbm.at[pl.ds(off, C)], idx_v)
        pltpu.async_copy(table_hbm.at[idx_v], rows_v, gsem).wait()

        @pl.loop(0, C)
        def _scale(i):
            rows_v[i, pl.ds(0, 16)] = rows_v[i, pl.ds(0, 16)] * SCALE
            rows_v[i, pl.ds(16, 16)] = rows_v[i, pl.ds(16, 16)] * SCALE

        pltpu.sync_copy(rows_v, out_hbm.at[pl.ds(off, C)])


@jax.jit
def _embed(x_flat, table):
    mesh = plsc.VectorSubcoreMesh(
        core_axis_name="c", subcore_axis_name="s", num_cores=NC, num_subcores=NS
    )
    return pl.kernel(
        _body,
        out_type=jax.ShapeDtypeStruct((B, D), jnp.float32),
        mesh=mesh,
        scratch_types=[
            pltpu.VMEM((C,), jnp.int32),
            pltpu.VMEM((C, D), jnp.float32),
            pltpu.SemaphoreType.DMA,
        ],
    )(x_flat, table)


def kernel(x, table):
    out = _embed(x.reshape(B).astype(jnp.int32), table)
    return out.reshape(ROWS, COLS, D)


# speedup vs baseline: 4.0411x; 4.0411x over previous
import jax
import jax.numpy as jnp
from jax import lax
from jax.experimental import pallas as pl
from jax.experimental.pallas import tpu as pltpu
from jax.experimental.pallas import tpu_sc as plsc

VOCAB = 1000000
D = 32
ROWS = 16384
COLS = 200
B = ROWS * COLS
NC = 2
NS = 16
NW = NC * NS
B_PER_W = B // NW
C = 1024
NCHUNK = B_PER_W // C
SCALE = float(D) ** 0.5


def _body(x_hbm, table_hbm, out_hbm, idx_v, rows_v, gsem):
    wid = lax.axis_index("s") * NC + lax.axis_index("c")
    base = wid * B_PER_W

    @pl.loop(0, NCHUNK)
    def _chunk(g):
        off = base + g * C
        pltpu.sync_copy(x_hbm.at[pl.ds(off, C)], idx_v)
        pltpu.async_copy(table_hbm.at[idx_v], rows_v, gsem).wait()

        @pl.loop(0, C)
        def _scale(i):
            rows_v[i, pl.ds(0, 16)] = rows_v[i, pl.ds(0, 16)] * SCALE
            rows_v[i, pl.ds(16, 16)] = rows_v[i, pl.ds(16, 16)] * SCALE

        pltpu.sync_copy(rows_v, out_hbm.at[pl.ds(off, C)])


@jax.jit
def _embed(x_flat, table):
    mesh = plsc.VectorSubcoreMesh(
        core_axis_name="c", subcore_axis_name="s", num_cores=NC, num_subcores=NS
    )
    return pl.kernel(
        _body,
        out_type=jax.ShapeDtypeStruct((B, D), jnp.float32),
        mesh=mesh,
        compiler_params=pltpu.CompilerParams(use_tc_tiling_on_sc=False),
        scratch_types=[
            pltpu.VMEM((C,), jnp.int32),
            pltpu.VMEM((C, D), jnp.float32),
            pltpu.SemaphoreType.DMA,
        ],
    )(x_flat, table)


def kernel(x, table):
    out = _embed(x.reshape(B).astype(jnp.int32), table)
    return out.reshape(ROWS, COLS, D)


# speedup vs baseline: 4.9700x; 1.2299x over previous
import jax
import jax.numpy as jnp
from jax import lax
from jax.experimental import pallas as pl
from jax.experimental.pallas import tpu as pltpu
from jax.experimental.pallas import tpu_sc as plsc

VOCAB = 1000000
D = 32
ROWS = 16384
COLS = 200
B = ROWS * COLS
NC = 2
NS = 16
NW = NC * NS
B_PER_W = B // NW
C = 1600
NCHUNK = B_PER_W // C
SCALE = float(D) ** 0.5


def _body(x_hbm, table_hbm, out_hbm, idx_v, rows_v, gsem, wsem):
    wid = lax.axis_index("s") * NC + lax.axis_index("c")
    base = wid * B_PER_W

    def fetch(g, b):
        off = base + g * C
        pltpu.sync_copy(x_hbm.at[pl.ds(off, C)], idx_v.at[b])
        pltpu.async_copy(table_hbm.at[idx_v.at[b]], rows_v.at[b], gsem.at[b])

    fetch(0, 0)

    @pl.loop(0, NCHUNK)
    def _chunk(g):
        b = lax.rem(g, 2)
        nb = 1 - b
        off = base + g * C

        @pl.when(g + 1 < NCHUNK)
        def _():
            @pl.when(g >= 1)
            def _():
                pltpu.make_async_copy(
                    rows_v.at[nb], out_hbm.at[pl.ds(base, C)], wsem.at[nb]
                ).wait()

            fetch(g + 1, nb)

        pltpu.make_async_copy(
            table_hbm.at[idx_v.at[b]], rows_v.at[b], gsem.at[b]
        ).wait()

        rv = rows_v.at[b]

        @pl.loop(0, C, unroll=8)
        def _scale(i):
            rv[i, pl.ds(0, 16)] = rv[i, pl.ds(0, 16)] * SCALE
            rv[i, pl.ds(16, 16)] = rv[i, pl.ds(16, 16)] * SCALE

        pltpu.async_copy(rv, out_hbm.at[pl.ds(off, C)], wsem.at[b])

    pltpu.make_async_copy(rows_v.at[0], out_hbm.at[pl.ds(base, C)], wsem.at[0]).wait()
    pltpu.make_async_copy(rows_v.at[1], out_hbm.at[pl.ds(base, C)], wsem.at[1]).wait()


@jax.jit
def _embed(x_flat, table):
    mesh = plsc.VectorSubcoreMesh(
        core_axis_name="c", subcore_axis_name="s", num_cores=NC, num_subcores=NS
    )
    return pl.kernel(
        _body,
        out_type=jax.ShapeDtypeStruct((B, D), jnp.float32),
        mesh=mesh,
        compiler_params=pltpu.CompilerParams(use_tc_tiling_on_sc=False),
        scratch_types=[
            pltpu.VMEM((2, C), jnp.int32),
            pltpu.VMEM((2, C, D), jnp.float32),
            pltpu.SemaphoreType.DMA((2,)),
            pltpu.SemaphoreType.DMA((2,)),
        ],
    )(x_flat, table)


def kernel(x, table):
    out = _embed(x.reshape(B).astype(jnp.int32), table)
    return out.reshape(ROWS, COLS, D)
